# Initial kernel scaffold; baseline (speedup 1.0000x reference)
import jax
import jax.numpy as jnp
from jax.experimental import pallas as pl


def kernel(node_types, node_labels, edge_types, edge_labels, edge_index, W_self0, W_neigh0, W_edge0, b0, W_self1, W_neigh1, W_edge1, b1, W_fc, b_fc):
    raise NotImplementedError("write your pallas kernel here")


# speedup vs baseline: 7.4379x; 7.4379x over previous
import functools

import jax
import jax.numpy as jnp
from jax import lax
from jax.experimental import pallas as pl
from jax.experimental.pallas import tpu as pltpu
from jax.experimental.pallas import tpu_sc as plsc

NC = 2
NS = 16
NW = NC * NS
CH = 128
WB = 1250


def _seg_sum_nodes(src, dst, h, zeros16, n_nodes, n_edges):
  n_chunks = n_edges // CH
  iters = (n_chunks + NW - 1) // NW
  zr = n_nodes // NS

  mesh = plsc.VectorSubcoreMesh(core_axis_name="c", subcore_axis_name="s")

  @functools.partial(
      pl.kernel,
      out_type=jax.ShapeDtypeStruct((NC, n_nodes, 16), jnp.float32),
      mesh=mesh,
      compiler_params=pltpu.CompilerParams(use_tc_tiling_on_sc=False),
      scratch_types=[
          pltpu.VMEM((CH,), jnp.int32),
          pltpu.VMEM((CH,), jnp.int32),
          pltpu.VMEM((CH, 16), jnp.float32),
          pltpu.VMEM((WB, 16), jnp.float32),
          pltpu.VMEM_SHARED((n_nodes, 16), jnp.float32),
          pltpu.SemaphoreType.DMA,
      ],
  )
  def seg_kernel(src_hbm, dst_hbm, h_hbm, z_hbm, out_hbm,
                 src_v, dst_v, rows_v, wb_v, acc_sh, sem):
    c = lax.axis_index("c")
    s = lax.axis_index("s")
    w = s * NC + c

    pltpu.sync_copy(z_hbm, wb_v)
    for k in range(zr // WB):
      pltpu.sync_copy(wb_v, acc_sh.at[pl.ds(s * zr + k * WB, WB), :])
    plsc.subcore_barrier()

    def body(i, carry):
      chunk = i * NW + w

      @pl.when(chunk < n_chunks)
      def _():
        e0 = chunk * CH
        pltpu.sync_copy(src_hbm.at[pl.ds(e0, CH)], src_v)
        pltpu.sync_copy(dst_hbm.at[pl.ds(e0, CH)], dst_v)
        pltpu.async_copy(h_hbm.at[src_v], rows_v, sem).wait()
        pltpu.sync_copy(rows_v, acc_sh.at[dst_v], add=True)

      return carry

    lax.fori_loop(0, iters, body, 0)
    plsc.subcore_barrier()

    for k in range(zr // WB):
      r0 = s * zr + k * WB
      pltpu.sync_copy(acc_sh.at[pl.ds(r0, WB), :], wb_v)
      pltpu.sync_copy(wb_v, out_hbm.at[c, pl.ds(r0, WB), :])

  return seg_kernel(src, dst, h, zeros16)


def _seg_sum_edges(dst, et, el, zeros8, n_nodes, n_edges):
  n_chunks = n_edges // CH
  iters = (n_chunks + NW - 1) // NW
  zr = n_nodes // NS

  mesh = plsc.VectorSubcoreMesh(core_axis_name="c", subcore_axis_name="s")

  @functools.partial(
      pl.kernel,
      out_type=(jax.ShapeDtypeStruct((NC, n_nodes, 8), jnp.float32),
                jax.ShapeDtypeStruct((NC, n_nodes, 8), jnp.float32)),
      mesh=mesh,
      compiler_params=pltpu.CompilerParams(use_tc_tiling_on_sc=False),
      scratch_types=[
          pltpu.VMEM((CH,), jnp.int32),
          pltpu.VMEM((CH, 8), jnp.float32),
          pltpu.VMEM((CH, 8), jnp.float32),
          pltpu.VMEM((WB, 8), jnp.float32),
          pltpu.VMEM_SHARED((n_nodes, 8), jnp.float32),
          pltpu.VMEM_SHARED((n_nodes, 8), jnp.float32),
      ],
  )
  def ef_kernel(dst_hbm, et_hbm, el_hbm, z_hbm, outT_hbm, outL_hbm,
                dst_v, et_v, el_v, wb_v, accT_sh, accL_sh):
    c = lax.axis_index("c")
    s = lax.axis_index("s")
    w = s * NC + c

    pltpu.sync_copy(z_hbm, wb_v)
    for k in range(zr // WB):
      r0 = s * zr + k * WB
      pltpu.sync_copy(wb_v, accT_sh.at[pl.ds(r0, WB), :])
      pltpu.sync_copy(wb_v, accL_sh.at[pl.ds(r0, WB), :])
    plsc.subcore_barrier()

    def body(i, carry):
      chunk = i * NW + w

      @pl.when(chunk < n_chunks)
      def _():
        e0 = chunk * CH
        pltpu.sync_copy(dst_hbm.at[pl.ds(e0, CH)], dst_v)
        pltpu.sync_copy(et_hbm.at[pl.ds(e0, CH), :], et_v)
        pltpu.sync_copy(el_hbm.at[pl.ds(e0, CH), :], el_v)
        pltpu.sync_copy(et_v, accT_sh.at[dst_v], add=True)
        pltpu.sync_copy(el_v, accL_sh.at[dst_v], add=True)

      return carry

    lax.fori_loop(0, iters, body, 0)
    plsc.subcore_barrier()

    for k in range(zr // WB):
      r0 = s * zr + k * WB
      pltpu.sync_copy(accT_sh.at[pl.ds(r0, WB), :], wb_v)
      pltpu.sync_copy(wb_v, outT_hbm.at[c, pl.ds(r0, WB), :])
      pltpu.sync_copy(accL_sh.at[pl.ds(r0, WB), :], wb_v)
      pltpu.sync_copy(wb_v, outL_hbm.at[c, pl.ds(r0, WB), :])

  return ef_kernel(dst, et, el, zeros8)


BLK = 5000


def _dense_layer(h, agg_p, accT_p, accL_p, w_self, w_neigh, w_et, w_el, b):
  n = h.shape[0]
  grid = n // BLK

  def body(h_ref, agg_ref, t_ref, l_ref, ws_ref, wn_ref, wt_ref, wl_ref,
           b_ref, out_ref):
    agg = agg_ref[0] + agg_ref[1]
    eft = t_ref[0] + t_ref[1]
    efl = l_ref[0] + l_ref[1]
    acc = jnp.dot(h_ref[...], ws_ref[...], preferred_element_type=jnp.float32)
    acc = acc + jnp.dot(agg, wn_ref[...], preferred_element_type=jnp.float32)
    acc = acc + jnp.dot(eft, wt_ref[...], preferred_element_type=jnp.float32)
    acc = acc + jnp.dot(efl, wl_ref[...], preferred_element_type=jnp.float32)
    out_ref[...] = jnp.maximum(acc + b_ref[...], 0.0)

  return pl.pallas_call(
      body,
      grid=(grid,),
      in_specs=[
          pl.BlockSpec((BLK, 16), lambda i: (i, 0)),
          pl.BlockSpec((NC, BLK, 16), lambda i: (0, i, 0)),
          pl.BlockSpec((NC, BLK, 8), lambda i: (0, i, 0)),
          pl.BlockSpec((NC, BLK, 8), lambda i: (0, i, 0)),
          pl.BlockSpec((16, 16), lambda i: (0, 0)),
          pl.BlockSpec((16, 16), lambda i: (0, 0)),
          pl.BlockSpec((8, 16), lambda i: (0, 0)),
          pl.BlockSpec((8, 16), lambda i: (0, 0)),
          pl.BlockSpec((1, 16), lambda i: (0, 0)),
      ],
      out_specs=pl.BlockSpec((BLK, 16), lambda i: (i, 0)),
      out_shape=jax.ShapeDtypeStruct((n, 16), jnp.float32),
  )(h, agg_p, accT_p, accL_p, w_self, w_neigh, w_et, w_el, b)


def _dense_layer_pooled(h, agg_p, accT_p, accL_p, w_self, w_neigh, w_et,
                        w_el, b):
  n = h.shape[0]
  grid = n // BLK

  def body(h_ref, agg_ref, t_ref, l_ref, ws_ref, wn_ref, wt_ref, wl_ref,
           b_ref, out_ref):
    i = pl.program_id(0)
    agg = agg_ref[0] + agg_ref[1]
    eft = t_ref[0] + t_ref[1]
    efl = l_ref[0] + l_ref[1]
    acc = jnp.dot(h_ref[...], ws_ref[...], preferred_element_type=jnp.float32)
    acc = acc + jnp.dot(agg, wn_ref[...], preferred_element_type=jnp.float32)
    acc = acc + jnp.dot(eft, wt_ref[...], preferred_element_type=jnp.float32)
    acc = acc + jnp.dot(efl, wl_ref[...], preferred_element_type=jnp.float32)
    h2 = jnp.maximum(acc + b_ref[...], 0.0)
    part = jnp.sum(h2.reshape(BLK // 8, 8, 16), axis=0)

    @pl.when(i == 0)
    def _():
      out_ref[...] = jnp.zeros_like(out_ref)

    out_ref[...] += part

  return pl.pallas_call(
      body,
      grid=(grid,),
      in_specs=[
          pl.BlockSpec((BLK, 16), lambda i: (i, 0)),
          pl.BlockSpec((NC, BLK, 16), lambda i: (0, i, 0)),
          pl.BlockSpec((NC, BLK, 8), lambda i: (0, i, 0)),
          pl.BlockSpec((NC, BLK, 8), lambda i: (0, i, 0)),
          pl.BlockSpec((16, 16), lambda i: (0, 0)),
          pl.BlockSpec((16, 16), lambda i: (0, 0)),
          pl.BlockSpec((8, 16), lambda i: (0, 0)),
          pl.BlockSpec((8, 16), lambda i: (0, 0)),
          pl.BlockSpec((1, 16), lambda i: (0, 0)),
      ],
      out_specs=pl.BlockSpec((8, 16), lambda i: (0, 0)),
      out_shape=jax.ShapeDtypeStruct((8, 16), jnp.float32),
  )(h, agg_p, accT_p, accL_p, w_self, w_neigh, w_et, w_el, b)


def kernel(node_types, node_labels, edge_types, edge_labels, edge_index,
           W_self0, W_neigh0, W_edge0, b0,
           W_self1, W_neigh1, W_edge1, b1,
           W_fc, b_fc):
  n = node_types.shape[0]
  e = edge_types.shape[0]
  src = edge_index[0]
  dst = edge_index[1]

  h0 = jnp.concatenate([node_types * (1.0 / 5.0), node_labels], axis=1)
  w_et0 = W_edge0[:8] * (1.0 / 5.0)
  w_el0 = W_edge0[8:]
  w_et1 = W_edge1[:8] * (1.0 / 5.0)
  w_el1 = W_edge1[8:]
  b0r = b0.reshape(1, 16)
  b1r = b1.reshape(1, 16)

  zeros16 = jnp.zeros((WB, 16), jnp.float32)
  zeros8 = jnp.zeros((WB, 8), jnp.float32)

  accT_p, accL_p = _seg_sum_edges(dst, edge_types, edge_labels, zeros8, n, e)
  agg0_p = _seg_sum_nodes(src, dst, h0, zeros16, n, e)
  h1 = _dense_layer(h0, agg0_p, accT_p, accL_p, W_self0, W_neigh0,
                    w_et0, w_el0, b0r)
  agg1_p = _seg_sum_nodes(src, dst, h1, zeros16, n, e)
  pooled8 = _dense_layer_pooled(h1, agg1_p, accT_p, accL_p, W_self1,
                                W_neigh1, w_et1, w_el1, b1r)

  pooled = jnp.sum(pooled8, axis=0, keepdims=True)
  return pooled @ W_fc + b_fc


# speedup vs baseline: 11.2109x; 1.5073x over previous
import functools

import jax
import jax.numpy as jnp
from jax import lax
from jax.experimental import pallas as pl
from jax.experimental.pallas import tpu as pltpu
from jax.experimental.pallas import tpu_sc as plsc

NC = 2
NS = 16
NW = NC * NS
CH = 128
WB = 250


KB = 4


def _seg_sum_nodes(src2, dst2, h, zeros16, n_nodes, n_edges):
  n_blocks = n_edges // (CH * KB)
  iters = (n_blocks + NW - 1) // NW
  zr = n_nodes // NS

  mesh = plsc.VectorSubcoreMesh(core_axis_name="c", subcore_axis_name="s")

  @functools.partial(
      pl.kernel,
      out_type=jax.ShapeDtypeStruct((NC, n_nodes, 16), jnp.float32),
      mesh=mesh,
      compiler_params=pltpu.CompilerParams(use_tc_tiling_on_sc=False),
      scratch_types=[
          pltpu.VMEM((2, KB, CH), jnp.int32),
          pltpu.VMEM((2, KB, CH), jnp.int32),
          pltpu.VMEM((2, KB, CH, 16), jnp.float32),
          pltpu.VMEM((WB, 16), jnp.float32),
          pltpu.VMEM_SHARED((n_nodes, 16), jnp.float32),
          pltpu.SemaphoreType.DMA,
          pltpu.SemaphoreType.DMA,
      ],
  )
  def seg_kernel(src_hbm, dst_hbm, h_hbm, z_hbm, out_hbm,
                 src_v, dst_v, rows_v, wb_v, acc_sh, sem_g, sem_s):
    c = lax.axis_index("c")
    s = lax.axis_index("s")
    w = s * NC + c

    pltpu.sync_copy(z_hbm, wb_v)
    for k in range(zr // WB):
      pltpu.sync_copy(wb_v, acc_sh.at[pl.ds(s * zr + k * WB, WB), :])
    plsc.subcore_barrier()

    def body(i, carry):
      b = i * NW + w
      p = i % 2

      @pl.when(b < n_blocks)
      def _():
        pltpu.sync_copy(src_hbm.at[pl.ds(b * KB, KB), :], src_v.at[p])
        pltpu.sync_copy(dst_hbm.at[pl.ds(b * KB, KB), :], dst_v.at[p])
        for j in range(KB):
          pltpu.async_copy(h_hbm.at[src_v.at[p, j]], rows_v.at[p, j], sem_g)
        @pl.when(i >= 1)
        def _():
          for j in range(KB):
            pltpu.make_async_copy(rows_v.at[1 - p, j],
                                  acc_sh.at[dst_v.at[1 - p, j]], sem_s).wait()
        for j in range(KB):
          pltpu.make_async_copy(h_hbm.at[src_v.at[p, j]], rows_v.at[p, j],
                                sem_g).wait()
        for j in range(KB):
          pltpu.async_copy(rows_v.at[p, j], acc_sh.at[dst_v.at[p, j]],
                           sem_s, add=True)

      return carry

    lax.fori_loop(0, iters, body, 0)
    for j in range(KB):
      pltpu.make_async_copy(rows_v.at[0, j], acc_sh.at[dst_v.at[0, j]],
                            sem_s).wait()
    plsc.subcore_barrier()

    for k in range(zr // WB):
      r0 = s * zr + k * WB
      pltpu.sync_copy(acc_sh.at[pl.ds(r0, WB), :], wb_v)
      pltpu.sync_copy(wb_v, out_hbm.at[c, pl.ds(r0, WB), :])

  return seg_kernel(src2, dst2, h, zeros16)


def _seg_sum_edges(dst2, et, el, zeros8, n_nodes, n_edges):
  n_blocks = n_edges // (CH * KB)
  iters = (n_blocks + NW - 1) // NW
  zr = n_nodes // NS

  mesh = plsc.VectorSubcoreMesh(core_axis_name="c", subcore_axis_name="s")

  @functools.partial(
      pl.kernel,
      out_type=(jax.ShapeDtypeStruct((NC, n_nodes, 8), jnp.float32),
                jax.ShapeDtypeStruct((NC, n_nodes, 8), jnp.float32)),
      mesh=mesh,
      compiler_params=pltpu.CompilerParams(use_tc_tiling_on_sc=False),
      scratch_types=[
          pltpu.VMEM((2, KB, CH), jnp.int32),
          pltpu.VMEM((2, KB * CH, 8), jnp.float32),
          pltpu.VMEM((2, KB * CH, 8), jnp.float32),
          pltpu.VMEM((WB, 8), jnp.float32),
          pltpu.VMEM_SHARED((n_nodes, 8), jnp.float32),
          pltpu.VMEM_SHARED((n_nodes, 8), jnp.float32),
          pltpu.SemaphoreType.DMA,
      ],
  )
  def ef_kernel(dst_hbm, et_hbm, el_hbm, z_hbm, outT_hbm, outL_hbm,
                dst_v, et_v, el_v, wb_v, accT_sh, accL_sh, sem_s):
    c = lax.axis_index("c")
    s = lax.axis_index("s")
    w = s * NC + c

    pltpu.sync_copy(z_hbm, wb_v)
    for k in range(zr // WB):
      r0 = s * zr + k * WB
      pltpu.sync_copy(wb_v, accT_sh.at[pl.ds(r0, WB), :])
      pltpu.sync_copy(wb_v, accL_sh.at[pl.ds(r0, WB), :])
    plsc.subcore_barrier()

    def body(i, carry):
      b = i * NW + w
      p = i % 2

      @pl.when(b < n_blocks)
      def _():
        e0 = b * KB * CH
        pltpu.sync_copy(dst_hbm.at[pl.ds(b * KB, KB), :], dst_v.at[p])
        pltpu.sync_copy(et_hbm.at[pl.ds(e0, KB * CH), :], et_v.at[p])
        pltpu.sync_copy(el_hbm.at[pl.ds(e0, KB * CH), :], el_v.at[p])

        @pl.when(i >= 1)
        def _():
          for j in range(KB):
            pltpu.make_async_copy(et_v.at[1 - p, pl.ds(j * CH, CH), :],
                                  accT_sh.at[dst_v.at[1 - p, j]],
                                  sem_s).wait()
            pltpu.make_async_copy(el_v.at[1 - p, pl.ds(j * CH, CH), :],
                                  accL_sh.at[dst_v.at[1 - p, j]],
                                  sem_s).wait()
        for j in range(KB):
          pltpu.async_copy(et_v.at[p, pl.ds(j * CH, CH), :],
                           accT_sh.at[dst_v.at[p, j]], sem_s, add=True)
          pltpu.async_copy(el_v.at[p, pl.ds(j * CH, CH), :],
                           accL_sh.at[dst_v.at[p, j]], sem_s, add=True)

      return carry

    lax.fori_loop(0, iters, body, 0)
    for j in range(KB):
      pltpu.make_async_copy(et_v.at[0, pl.ds(j * CH, CH), :],
                            accT_sh.at[dst_v.at[0, j]], sem_s).wait()
      pltpu.make_async_copy(el_v.at[0, pl.ds(j * CH, CH), :],
                            accL_sh.at[dst_v.at[0, j]], sem_s).wait()
    plsc.subcore_barrier()

    for k in range(zr // WB):
      r0 = s * zr + k * WB
      pltpu.sync_copy(accT_sh.at[pl.ds(r0, WB), :], wb_v)
      pltpu.sync_copy(wb_v, outT_hbm.at[c, pl.ds(r0, WB), :])
      pltpu.sync_copy(accL_sh.at[pl.ds(r0, WB), :], wb_v)
      pltpu.sync_copy(wb_v, outL_hbm.at[c, pl.ds(r0, WB), :])

  return ef_kernel(dst2, et, el, zeros8)


BLK = 5000


def _dense_layer(h, agg_p, accT_p, accL_p, w_self, w_neigh, w_et, w_el, b):
  n = h.shape[0]
  grid = n // BLK

  def body(h_ref, agg_ref, t_ref, l_ref, ws_ref, wn_ref, wt_ref, wl_ref,
           b_ref, out_ref):
    agg = agg_ref[0] + agg_ref[1]
    eft = t_ref[0] + t_ref[1]
    efl = l_ref[0] + l_ref[1]
    acc = jnp.dot(h_ref[...], ws_ref[...], preferred_element_type=jnp.float32)
    acc = acc + jnp.dot(agg, wn_ref[...], preferred_element_type=jnp.float32)
    acc = acc + jnp.dot(eft, wt_ref[...], preferred_element_type=jnp.float32)
    acc = acc + jnp.dot(efl, wl_ref[...], preferred_element_type=jnp.float32)
    out_ref[...] = jnp.maximum(acc + b_ref[...], 0.0)

  return pl.pallas_call(
      body,
      grid=(grid,),
      in_specs=[
          pl.BlockSpec((BLK, 16), lambda i: (i, 0)),
          pl.BlockSpec((NC, BLK, 16), lambda i: (0, i, 0)),
          pl.BlockSpec((NC, BLK, 8), lambda i: (0, i, 0)),
          pl.BlockSpec((NC, BLK, 8), lambda i: (0, i, 0)),
          pl.BlockSpec((16, 16), lambda i: (0, 0)),
          pl.BlockSpec((16, 16), lambda i: (0, 0)),
          pl.BlockSpec((8, 16), lambda i: (0, 0)),
          pl.BlockSpec((8, 16), lambda i: (0, 0)),
          pl.BlockSpec((1, 16), lambda i: (0, 0)),
      ],
      out_specs=pl.BlockSpec((BLK, 16), lambda i: (i, 0)),
      out_shape=jax.ShapeDtypeStruct((n, 16), jnp.float32),
  )(h, agg_p, accT_p, accL_p, w_self, w_neigh, w_et, w_el, b)


def _dense_layer_pooled(h, agg_p, accT_p, accL_p, w_self, w_neigh, w_et,
                        w_el, b):
  n = h.shape[0]
  grid = n // BLK

  def body(h_ref, agg_ref, t_ref, l_ref, ws_ref, wn_ref, wt_ref, wl_ref,
           b_ref, out_ref):
    i = pl.program_id(0)
    agg = agg_ref[0] + agg_ref[1]
    eft = t_ref[0] + t_ref[1]
    efl = l_ref[0] + l_ref[1]
    acc = jnp.dot(h_ref[...], ws_ref[...], preferred_element_type=jnp.float32)
    acc = acc + jnp.dot(agg, wn_ref[...], preferred_element_type=jnp.float32)
    acc = acc + jnp.dot(eft, wt_ref[...], preferred_element_type=jnp.float32)
    acc = acc + jnp.dot(efl, wl_ref[...], preferred_element_type=jnp.float32)
    h2 = jnp.maximum(acc + b_ref[...], 0.0)
    part = jnp.sum(h2.reshape(BLK // 8, 8, 16), axis=0)

    @pl.when(i == 0)
    def _():
      out_ref[...] = jnp.zeros_like(out_ref)

    out_ref[...] += part

  return pl.pallas_call(
      body,
      grid=(grid,),
      in_specs=[
          pl.BlockSpec((BLK, 16), lambda i: (i, 0)),
          pl.BlockSpec((NC, BLK, 16), lambda i: (0, i, 0)),
          pl.BlockSpec((NC, BLK, 8), lambda i: (0, i, 0)),
          pl.BlockSpec((NC, BLK, 8), lambda i: (0, i, 0)),
          pl.BlockSpec((16, 16), lambda i: (0, 0)),
          pl.BlockSpec((16, 16), lambda i: (0, 0)),
          pl.BlockSpec((8, 16), lambda i: (0, 0)),
          pl.BlockSpec((8, 16), lambda i: (0, 0)),
          pl.BlockSpec((1, 16), lambda i: (0, 0)),
      ],
      out_specs=pl.BlockSpec((8, 16), lambda i: (0, 0)),
      out_shape=jax.ShapeDtypeStruct((8, 16), jnp.float32),
  )(h, agg_p, accT_p, accL_p, w_self, w_neigh, w_et, w_el, b)


def kernel(node_types, node_labels, edge_types, edge_labels, edge_index,
           W_self0, W_neigh0, W_edge0, b0,
           W_self1, W_neigh1, W_edge1, b1,
           W_fc, b_fc):
  n = node_types.shape[0]
  e = edge_types.shape[0]
  src2 = edge_index[0].reshape(e // CH, CH)
  dst2 = edge_index[1].reshape(e // CH, CH)

  h0 = jnp.concatenate([node_types * (1.0 / 5.0), node_labels], axis=1)
  w_et0 = W_edge0[:8] * (1.0 / 5.0)
  w_el0 = W_edge0[8:]
  w_et1 = W_edge1[:8] * (1.0 / 5.0)
  w_el1 = W_edge1[8:]
  b0r = b0.reshape(1, 16)
  b1r = b1.reshape(1, 16)

  zeros16 = jnp.zeros((WB, 16), jnp.float32)
  zeros8 = jnp.zeros((WB, 8), jnp.float32)

  accT_p, accL_p = _seg_sum_edges(dst2, edge_types, edge_labels, zeros8, n, e)
  agg0_p = _seg_sum_nodes(src2, dst2, h0, zeros16, n, e)
  h1 = _dense_layer(h0, agg0_p, accT_p, accL_p, W_self0, W_neigh0,
                    w_et0, w_el0, b0r)
  agg1_p = _seg_sum_nodes(src2, dst2, h1, zeros16, n, e)
  pooled8 = _dense_layer_pooled(h1, agg1_p, accT_p, accL_p, W_self1,
                                W_neigh1, w_et1, w_el1, b1r)

  pooled = jnp.sum(pooled8, axis=0, keepdims=True)
  return pooled @ W_fc + b_fc


# speedup vs baseline: 11.2269x; 1.0014x over previous
import functools

import jax
import jax.numpy as jnp
from jax import lax
from jax.experimental import pallas as pl
from jax.experimental.pallas import tpu as pltpu
from jax.experimental.pallas import tpu_sc as plsc

NC = 2
NS = 16
NW = NC * NS
CH = 128
WB = 250


KB = 4


def _seg_sum_nodes(src2, dst2, h, zeros16, n_nodes, n_edges):
  n_blocks = n_edges // (CH * KB)
  iters = (n_blocks + NW - 1) // NW
  zr = n_nodes // NS

  mesh = plsc.VectorSubcoreMesh(core_axis_name="c", subcore_axis_name="s")

  @functools.partial(
      pl.kernel,
      out_type=jax.ShapeDtypeStruct((NC, n_nodes, 16), jnp.float32),
      mesh=mesh,
      compiler_params=pltpu.CompilerParams(use_tc_tiling_on_sc=False),
      scratch_types=[
          pltpu.VMEM((2, KB, CH), jnp.int32),
          pltpu.VMEM((2, KB, CH), jnp.int32),
          pltpu.VMEM((2, KB, CH, 16), jnp.float32),
          pltpu.VMEM((WB, 16), jnp.float32),
          pltpu.VMEM_SHARED((n_nodes, 16), jnp.float32),
          pltpu.SemaphoreType.DMA,
          pltpu.SemaphoreType.DMA,
      ],
  )
  def seg_kernel(src_hbm, dst_hbm, h_hbm, z_hbm, out_hbm,
                 src_v, dst_v, rows_v, wb_v, acc_sh, sem_g, sem_s):
    c = lax.axis_index("c")
    s = lax.axis_index("s")
    w = s * NC + c

    pltpu.sync_copy(z_hbm, wb_v)
    for k in range(zr // WB):
      pltpu.sync_copy(wb_v, acc_sh.at[pl.ds(s * zr + k * WB, WB), :])
    plsc.subcore_barrier()

    def body(i, carry):
      b = i * NW + w
      p = i % 2

      @pl.when(b < n_blocks)
      def _():
        pltpu.sync_copy(src_hbm.at[pl.ds(b * KB, KB), :], src_v.at[p])
        pltpu.sync_copy(dst_hbm.at[pl.ds(b * KB, KB), :], dst_v.at[p])
        for j in range(KB):
          pltpu.async_copy(h_hbm.at[src_v.at[p, j]], rows_v.at[p, j], sem_g)
        @pl.when(i >= 1)
        def _():
          for j in range(KB):
            pltpu.make_async_copy(rows_v.at[1 - p, j],
                                  acc_sh.at[dst_v.at[1 - p, j]], sem_s).wait()
        for j in range(KB):
          pltpu.make_async_copy(h_hbm.at[src_v.at[p, j]], rows_v.at[p, j],
                                sem_g).wait()
        for j in range(KB):
          pltpu.async_copy(rows_v.at[p, j], acc_sh.at[dst_v.at[p, j]],
                           sem_s, add=True)

      return carry

    lax.fori_loop(0, iters, body, 0)
    for j in range(KB):
      pltpu.make_async_copy(rows_v.at[0, j], acc_sh.at[dst_v.at[0, j]],
                            sem_s).wait()
    plsc.subcore_barrier()

    for k in range(zr // WB):
      r0 = s * zr + k * WB
      pltpu.sync_copy(acc_sh.at[pl.ds(r0, WB), :], wb_v)
      pltpu.sync_copy(wb_v, out_hbm.at[c, pl.ds(r0, WB), :])

  return seg_kernel(src2, dst2, h, zeros16)


def _seg_sum_edges(dst2, et, el, zeros8, n_nodes, n_edges):
  n_blocks = n_edges // (CH * KB)
  iters = (n_blocks + NW - 1) // NW
  zr = n_nodes // NS

  mesh = plsc.VectorSubcoreMesh(core_axis_name="c", subcore_axis_name="s")

  @functools.partial(
      pl.kernel,
      out_type=(jax.ShapeDtypeStruct((NC, n_nodes, 8), jnp.float32),
                jax.ShapeDtypeStruct((NC, n_nodes, 8), jnp.float32)),
      mesh=mesh,
      compiler_params=pltpu.CompilerParams(use_tc_tiling_on_sc=False),
      scratch_types=[
          pltpu.VMEM((2, KB, CH), jnp.int32),
          pltpu.VMEM((2, KB * CH, 8), jnp.float32),
          pltpu.VMEM((2, KB * CH, 8), jnp.float32),
          pltpu.VMEM((WB, 8), jnp.float32),
          pltpu.VMEM_SHARED((n_nodes, 8), jnp.float32),
          pltpu.VMEM_SHARED((n_nodes, 8), jnp.float32),
          pltpu.SemaphoreType.DMA,
      ],
  )
  def ef_kernel(dst_hbm, et_hbm, el_hbm, z_hbm, outT_hbm, outL_hbm,
                dst_v, et_v, el_v, wb_v, accT_sh, accL_sh, sem_s):
    c = lax.axis_index("c")
    s = lax.axis_index("s")
    w = s * NC + c

    pltpu.sync_copy(z_hbm, wb_v)
    for k in range(zr // WB):
      r0 = s * zr + k * WB
      pltpu.sync_copy(wb_v, accT_sh.at[pl.ds(r0, WB), :])
      pltpu.sync_copy(wb_v, accL_sh.at[pl.ds(r0, WB), :])
    plsc.subcore_barrier()

    def body(i, carry):
      b = i * NW + w
      p = i % 2

      @pl.when(b < n_blocks)
      def _():
        e0 = b * KB * CH
        pltpu.sync_copy(dst_hbm.at[pl.ds(b * KB, KB), :], dst_v.at[p])
        pltpu.sync_copy(et_hbm.at[pl.ds(e0, KB * CH), :], et_v.at[p])
        pltpu.sync_copy(el_hbm.at[pl.ds(e0, KB * CH), :], el_v.at[p])

        @pl.when(i >= 1)
        def _():
          for j in range(KB):
            pltpu.make_async_copy(et_v.at[1 - p, pl.ds(j * CH, CH), :],
                                  accT_sh.at[dst_v.at[1 - p, j]],
                                  sem_s).wait()
            pltpu.make_async_copy(el_v.at[1 - p, pl.ds(j * CH, CH), :],
                                  accL_sh.at[dst_v.at[1 - p, j]],
                                  sem_s).wait()
        for j in range(KB):
          pltpu.async_copy(et_v.at[p, pl.ds(j * CH, CH), :],
                           accT_sh.at[dst_v.at[p, j]], sem_s, add=True)
          pltpu.async_copy(el_v.at[p, pl.ds(j * CH, CH), :],
                           accL_sh.at[dst_v.at[p, j]], sem_s, add=True)

      return carry

    lax.fori_loop(0, iters, body, 0)
    for j in range(KB):
      pltpu.make_async_copy(et_v.at[0, pl.ds(j * CH, CH), :],
                            accT_sh.at[dst_v.at[0, j]], sem_s).wait()
      pltpu.make_async_copy(el_v.at[0, pl.ds(j * CH, CH), :],
                            accL_sh.at[dst_v.at[0, j]], sem_s).wait()
    plsc.subcore_barrier()

    for k in range(zr // WB):
      r0 = s * zr + k * WB
      pltpu.sync_copy(accT_sh.at[pl.ds(r0, WB), :], wb_v)
      pltpu.sync_copy(wb_v, outT_hbm.at[c, pl.ds(r0, WB), :])
      pltpu.sync_copy(accL_sh.at[pl.ds(r0, WB), :], wb_v)
      pltpu.sync_copy(wb_v, outL_hbm.at[c, pl.ds(r0, WB), :])

  return ef_kernel(dst2, et, el, zeros8)


def _split_edge_index(ei):
  e = ei.shape[1]
  blke = 128000
  grid = e // blke
  rb = blke // 128

  def body(ei_ref, s_ref, d_ref):
    x = ei_ref[...]
    s_ref[...] = x[0].reshape(rb, 128)
    d_ref[...] = x[1].reshape(rb, 128)

  return pl.pallas_call(
      body,
      grid=(grid,),
      in_specs=[pl.BlockSpec((2, blke), lambda i: (0, i))],
      out_specs=(pl.BlockSpec((rb, 128), lambda i: (i, 0)),
                 pl.BlockSpec((rb, 128), lambda i: (i, 0))),
      out_shape=(jax.ShapeDtypeStruct((e // 128, 128), jnp.int32),
                 jax.ShapeDtypeStruct((e // 128, 128), jnp.int32)),
  )(ei)


BLK = 5000


def _dense_layer(h, agg_p, accT_p, accL_p, w_self, w_neigh, w_et, w_el, b):
  n = h.shape[0]
  grid = n // BLK

  def body(h_ref, agg_ref, t_ref, l_ref, ws_ref, wn_ref, wt_ref, wl_ref,
           b_ref, out_ref):
    agg = agg_ref[0] + agg_ref[1]
    eft = t_ref[0] + t_ref[1]
    efl = l_ref[0] + l_ref[1]
    acc = jnp.dot(h_ref[...], ws_ref[...], preferred_element_type=jnp.float32)
    acc = acc + jnp.dot(agg, wn_ref[...], preferred_element_type=jnp.float32)
    acc = acc + jnp.dot(eft, wt_ref[...], preferred_element_type=jnp.float32)
    acc = acc + jnp.dot(efl, wl_ref[...], preferred_element_type=jnp.float32)
    out_ref[...] = jnp.maximum(acc + b_ref[...], 0.0)

  return pl.pallas_call(
      body,
      grid=(grid,),
      in_specs=[
          pl.BlockSpec((BLK, 16), lambda i: (i, 0)),
          pl.BlockSpec((NC, BLK, 16), lambda i: (0, i, 0)),
          pl.BlockSpec((NC, BLK, 8), lambda i: (0, i, 0)),
          pl.BlockSpec((NC, BLK, 8), lambda i: (0, i, 0)),
          pl.BlockSpec((16, 16), lambda i: (0, 0)),
          pl.BlockSpec((16, 16), lambda i: (0, 0)),
          pl.BlockSpec((8, 16), lambda i: (0, 0)),
          pl.BlockSpec((8, 16), lambda i: (0, 0)),
          pl.BlockSpec((1, 16), lambda i: (0, 0)),
      ],
      out_specs=pl.BlockSpec((BLK, 16), lambda i: (i, 0)),
      out_shape=jax.ShapeDtypeStruct((n, 16), jnp.float32),
  )(h, agg_p, accT_p, accL_p, w_self, w_neigh, w_et, w_el, b)


def _dense_layer_pooled(h, agg_p, accT_p, accL_p, w_self, w_neigh, w_et,
                        w_el, b):
  n = h.shape[0]
  grid = n // BLK

  def body(h_ref, agg_ref, t_ref, l_ref, ws_ref, wn_ref, wt_ref, wl_ref,
           b_ref, out_ref):
    i = pl.program_id(0)
    agg = agg_ref[0] + agg_ref[1]
    eft = t_ref[0] + t_ref[1]
    efl = l_ref[0] + l_ref[1]
    acc = jnp.dot(h_ref[...], ws_ref[...], preferred_element_type=jnp.float32)
    acc = acc + jnp.dot(agg, wn_ref[...], preferred_element_type=jnp.float32)
    acc = acc + jnp.dot(eft, wt_ref[...], preferred_element_type=jnp.float32)
    acc = acc + jnp.dot(efl, wl_ref[...], preferred_element_type=jnp.float32)
    h2 = jnp.maximum(acc + b_ref[...], 0.0)
    part = jnp.sum(h2.reshape(BLK // 8, 8, 16), axis=0)

    @pl.when(i == 0)
    def _():
      out_ref[...] = jnp.zeros_like(out_ref)

    out_ref[...] += part

  return pl.pallas_call(
      body,
      grid=(grid,),
      in_specs=[
          pl.BlockSpec((BLK, 16), lambda i: (i, 0)),
          pl.BlockSpec((NC, BLK, 16), lambda i: (0, i, 0)),
          pl.BlockSpec((NC, BLK, 8), lambda i: (0, i, 0)),
          pl.BlockSpec((NC, BLK, 8), lambda i: (0, i, 0)),
          pl.BlockSpec((16, 16), lambda i: (0, 0)),
          pl.BlockSpec((16, 16), lambda i: (0, 0)),
          pl.BlockSpec((8, 16), lambda i: (0, 0)),
          pl.BlockSpec((8, 16), lambda i: (0, 0)),
          pl.BlockSpec((1, 16), lambda i: (0, 0)),
      ],
      out_specs=pl.BlockSpec((8, 16), lambda i: (0, 0)),
      out_shape=jax.ShapeDtypeStruct((8, 16), jnp.float32),
  )(h, agg_p, accT_p, accL_p, w_self, w_neigh, w_et, w_el, b)


def kernel(node_types, node_labels, edge_types, edge_labels, edge_index,
           W_self0, W_neigh0, W_edge0, b0,
           W_self1, W_neigh1, W_edge1, b1,
           W_fc, b_fc):
  n = node_types.shape[0]
  e = edge_types.shape[0]
  src2, dst2 = _split_edge_index(edge_index)

  h0 = jnp.concatenate([node_types * (1.0 / 5.0), node_labels], axis=1)
  w_et0 = W_edge0[:8] * (1.0 / 5.0)
  w_el0 = W_edge0[8:]
  w_et1 = W_edge1[:8] * (1.0 / 5.0)
  w_el1 = W_edge1[8:]
  b0r = b0.reshape(1, 16)
  b1r = b1.reshape(1, 16)

  zeros16 = jnp.zeros((WB, 16), jnp.float32)
  zeros8 = jnp.zeros((WB, 8), jnp.float32)

  accT_p, accL_p = _seg_sum_edges(dst2, edge_types, edge_labels, zeros8, n, e)
  agg0_p = _seg_sum_nodes(src2, dst2, h0, zeros16, n, e)
  h1 = _dense_layer(h0, agg0_p, accT_p, accL_p, W_self0, W_neigh0,
                    w_et0, w_el0, b0r)
  agg1_p = _seg_sum_nodes(src2, dst2, h1, zeros16, n, e)
  pooled8 = _dense_layer_pooled(h1, agg1_p, accT_p, accL_p, W_self1,
                                W_neigh1, w_et1, w_el1, b1r)

  pooled = jnp.sum(pooled8, axis=0, keepdims=True)
  return pooled @ W_fc + b_fc


# speedup vs baseline: 12.3024x; 1.0958x over previous
import functools

import jax
import jax.numpy as jnp
from jax import lax
from jax.experimental import pallas as pl
from jax.experimental.pallas import tpu as pltpu
from jax.experimental.pallas import tpu_sc as plsc

NC = 2
NS = 16
NW = NC * NS
CH = 128
WB = 250


KB = 4


def _seg_sum_nodes(src2, dst2, h, zeros16, n_nodes, n_edges):
  n_blocks = n_edges // (CH * KB)
  iters = (n_blocks + NW - 1) // NW
  zr = n_nodes // NS

  mesh = plsc.VectorSubcoreMesh(core_axis_name="c", subcore_axis_name="s")

  @functools.partial(
      pl.kernel,
      out_type=jax.ShapeDtypeStruct((NC, n_nodes, 16), jnp.float32),
      mesh=mesh,
      compiler_params=pltpu.CompilerParams(use_tc_tiling_on_sc=False),
      scratch_types=[
          pltpu.VMEM((2, KB, CH), jnp.int32),
          pltpu.VMEM((2, KB, CH), jnp.int32),
          pltpu.VMEM((2, KB, CH, 16), jnp.float32),
          pltpu.VMEM((WB, 16), jnp.float32),
          pltpu.VMEM_SHARED((n_nodes, 16), jnp.float32),
          pltpu.SemaphoreType.DMA,
          pltpu.SemaphoreType.DMA,
      ],
  )
  def seg_kernel(src_hbm, dst_hbm, h_hbm, z_hbm, out_hbm,
                 src_v, dst_v, rows_v, wb_v, acc_sh, sem_g, sem_s):
    c = lax.axis_index("c")
    s = lax.axis_index("s")
    w = s * NC + c

    pltpu.sync_copy(z_hbm, wb_v)
    for k in range(zr // WB):
      pltpu.sync_copy(wb_v, acc_sh.at[pl.ds(s * zr + k * WB, WB), :])
    plsc.subcore_barrier()

    def body(i, carry):
      b = i * NW + w
      p = i % 2

      @pl.when(b < n_blocks)
      def _():
        pltpu.sync_copy(src_hbm.at[pl.ds(b * KB, KB), :], src_v.at[p])
        pltpu.sync_copy(dst_hbm.at[pl.ds(b * KB, KB), :], dst_v.at[p])
        for j in range(KB):
          pltpu.async_copy(h_hbm.at[src_v.at[p, j]], rows_v.at[p, j], sem_g)
        @pl.when(i >= 1)
        def _():
          for j in range(KB):
            pltpu.make_async_copy(rows_v.at[1 - p, j],
                                  acc_sh.at[dst_v.at[1 - p, j]], sem_s).wait()
        for j in range(KB):
          pltpu.make_async_copy(h_hbm.at[src_v.at[p, j]], rows_v.at[p, j],
                                sem_g).wait()
        for j in range(KB):
          pltpu.async_copy(rows_v.at[p, j], acc_sh.at[dst_v.at[p, j]],
                           sem_s, add=True)

      return carry

    lax.fori_loop(0, iters, body, 0)
    for j in range(KB):
      pltpu.make_async_copy(rows_v.at[0, j], acc_sh.at[dst_v.at[0, j]],
                            sem_s).wait()
    plsc.subcore_barrier()

    for k in range(zr // WB):
      r0 = s * zr + k * WB
      pltpu.sync_copy(acc_sh.at[pl.ds(r0, WB), :], wb_v)
      pltpu.sync_copy(wb_v, out_hbm.at[c, pl.ds(r0, WB), :])

  return seg_kernel(src2, dst2, h, zeros16)


def _pack_edge_features(etT, elT):
  e = etT.shape[1]
  blke = 12800
  grid = e // blke

  def body(t_ref, l_ref, o_ref):
    o_ref[...] = jnp.concatenate([t_ref[...].T, l_ref[...].T], axis=1)

  return pl.pallas_call(
      body,
      grid=(grid,),
      in_specs=[pl.BlockSpec((8, blke), lambda i: (0, i)),
                pl.BlockSpec((8, blke), lambda i: (0, i))],
      out_specs=pl.BlockSpec((blke, 16), lambda i: (i, 0)),
      out_shape=jax.ShapeDtypeStruct((e, 16), jnp.float32),
  )(etT, elT)


def _seg_sum_ef(dst2, ef, zeros16, n_nodes, n_edges):
  n_blocks = n_edges // (CH * KB)
  iters = (n_blocks + NW - 1) // NW
  zr = n_nodes // NS

  mesh = plsc.VectorSubcoreMesh(core_axis_name="c", subcore_axis_name="s")

  @functools.partial(
      pl.kernel,
      out_type=jax.ShapeDtypeStruct((NC, n_nodes, 16), jnp.float32),
      mesh=mesh,
      compiler_params=pltpu.CompilerParams(use_tc_tiling_on_sc=False),
      scratch_types=[
          pltpu.VMEM((2, KB, CH), jnp.int32),
          pltpu.VMEM((2, KB * CH, 16), jnp.float32),
          pltpu.VMEM((WB, 16), jnp.float32),
          pltpu.VMEM_SHARED((n_nodes, 16), jnp.float32),
          pltpu.SemaphoreType.DMA,
      ],
  )
  def ef_kernel(dst_hbm, ef_hbm, z_hbm, out_hbm,
                dst_v, ef_v, wb_v, acc_sh, sem_s):
    c = lax.axis_index("c")
    s = lax.axis_index("s")
    w = s * NC + c

    pltpu.sync_copy(z_hbm, wb_v)
    for k in range(zr // WB):
      pltpu.sync_copy(wb_v, acc_sh.at[pl.ds(s * zr + k * WB, WB), :])
    plsc.subcore_barrier()

    def body(i, carry):
      b = i * NW + w
      p = i % 2

      @pl.when(b < n_blocks)
      def _():
        e0 = b * KB * CH
        pltpu.sync_copy(dst_hbm.at[pl.ds(b * KB, KB), :], dst_v.at[p])
        pltpu.sync_copy(ef_hbm.at[pl.ds(e0, KB * CH), :], ef_v.at[p])

        @pl.when(i >= 1)
        def _():
          for j in range(KB):
            pltpu.make_async_copy(ef_v.at[1 - p, pl.ds(j * CH, CH), :],
                                  acc_sh.at[dst_v.at[1 - p, j]],
                                  sem_s).wait()
        for j in range(KB):
          pltpu.async_copy(ef_v.at[p, pl.ds(j * CH, CH), :],
                           acc_sh.at[dst_v.at[p, j]], sem_s, add=True)

      return carry

    lax.fori_loop(0, iters, body, 0)
    for j in range(KB):
      pltpu.make_async_copy(ef_v.at[0, pl.ds(j * CH, CH), :],
                            acc_sh.at[dst_v.at[0, j]], sem_s).wait()
    plsc.subcore_barrier()

    for k in range(zr // WB):
      r0 = s * zr + k * WB
      pltpu.sync_copy(acc_sh.at[pl.ds(r0, WB), :], wb_v)
      pltpu.sync_copy(wb_v, out_hbm.at[c, pl.ds(r0, WB), :])

  return ef_kernel(dst2, ef, zeros16)


def _split_edge_index(ei):
  e = ei.shape[1]
  blke = 128000
  grid = e // blke
  rb = blke // 128

  def body(ei_ref, s_ref, d_ref):
    x = ei_ref[...]
    s_ref[...] = x[0].reshape(rb, 128)
    d_ref[...] = x[1].reshape(rb, 128)

  return pl.pallas_call(
      body,
      grid=(grid,),
      in_specs=[pl.BlockSpec((2, blke), lambda i: (0, i))],
      out_specs=(pl.BlockSpec((rb, 128), lambda i: (i, 0)),
                 pl.BlockSpec((rb, 128), lambda i: (i, 0))),
      out_shape=(jax.ShapeDtypeStruct((e // 128, 128), jnp.int32),
                 jax.ShapeDtypeStruct((e // 128, 128), jnp.int32)),
  )(ei)


BLK = 5000


def _dense_layer(h, agg_p, ef_p, w_self, w_neigh, w_ef, b):
  n = h.shape[0]
  grid = n // BLK

  def body(h_ref, agg_ref, ef_ref, ws_ref, wn_ref, we_ref, b_ref, out_ref):
    agg = agg_ref[0] + agg_ref[1]
    ef = ef_ref[0] + ef_ref[1]
    acc = jnp.dot(h_ref[...], ws_ref[...], preferred_element_type=jnp.float32)
    acc = acc + jnp.dot(agg, wn_ref[...], preferred_element_type=jnp.float32)
    acc = acc + jnp.dot(ef, we_ref[...], preferred_element_type=jnp.float32)
    out_ref[...] = jnp.maximum(acc + b_ref[...], 0.0)

  return pl.pallas_call(
      body,
      grid=(grid,),
      in_specs=[
          pl.BlockSpec((BLK, 16), lambda i: (i, 0)),
          pl.BlockSpec((NC, BLK, 16), lambda i: (0, i, 0)),
          pl.BlockSpec((NC, BLK, 16), lambda i: (0, i, 0)),
          pl.BlockSpec((16, 16), lambda i: (0, 0)),
          pl.BlockSpec((16, 16), lambda i: (0, 0)),
          pl.BlockSpec((16, 16), lambda i: (0, 0)),
          pl.BlockSpec((1, 16), lambda i: (0, 0)),
      ],
      out_specs=pl.BlockSpec((BLK, 16), lambda i: (i, 0)),
      out_shape=jax.ShapeDtypeStruct((n, 16), jnp.float32),
  )(h, agg_p, ef_p, w_self, w_neigh, w_ef, b)


def _dense_layer_pooled(h, agg_p, ef_p, w_self, w_neigh, w_ef, b):
  n = h.shape[0]
  grid = n // BLK

  def body(h_ref, agg_ref, ef_ref, ws_ref, wn_ref, we_ref, b_ref, out_ref):
    i = pl.program_id(0)
    agg = agg_ref[0] + agg_ref[1]
    ef = ef_ref[0] + ef_ref[1]
    acc = jnp.dot(h_ref[...], ws_ref[...], preferred_element_type=jnp.float32)
    acc = acc + jnp.dot(agg, wn_ref[...], preferred_element_type=jnp.float32)
    acc = acc + jnp.dot(ef, we_ref[...], preferred_element_type=jnp.float32)
    h2 = jnp.maximum(acc + b_ref[...], 0.0)
    part = jnp.sum(h2.reshape(BLK // 8, 8, 16), axis=0)

    @pl.when(i == 0)
    def _():
      out_ref[...] = jnp.zeros_like(out_ref)

    out_ref[...] += part

  return pl.pallas_call(
      body,
      grid=(grid,),
      in_specs=[
          pl.BlockSpec((BLK, 16), lambda i: (i, 0)),
          pl.BlockSpec((NC, BLK, 16), lambda i: (0, i, 0)),
          pl.BlockSpec((NC, BLK, 16), lambda i: (0, i, 0)),
          pl.BlockSpec((16, 16), lambda i: (0, 0)),
          pl.BlockSpec((16, 16), lambda i: (0, 0)),
          pl.BlockSpec((16, 16), lambda i: (0, 0)),
          pl.BlockSpec((1, 16), lambda i: (0, 0)),
      ],
      out_specs=pl.BlockSpec((8, 16), lambda i: (0, 0)),
      out_shape=jax.ShapeDtypeStruct((8, 16), jnp.float32),
  )(h, agg_p, ef_p, w_self, w_neigh, w_ef, b)


def kernel(node_types, node_labels, edge_types, edge_labels, edge_index,
           W_self0, W_neigh0, W_edge0, b0,
           W_self1, W_neigh1, W_edge1, b1,
           W_fc, b_fc):
  n = node_types.shape[0]
  e = edge_types.shape[0]
  src2, dst2 = _split_edge_index(edge_index)

  h0 = jnp.concatenate([node_types * (1.0 / 5.0), node_labels], axis=1)
  scale = jnp.concatenate([jnp.full((8, 1), 1.0 / 5.0, jnp.float32),
                           jnp.ones((8, 1), jnp.float32)], axis=0)
  w_ef0 = W_edge0 * scale
  w_ef1 = W_edge1 * scale
  b0r = b0.reshape(1, 16)
  b1r = b1.reshape(1, 16)

  zeros16 = jnp.zeros((WB, 16), jnp.float32)

  ef = _pack_edge_features(edge_types.T, edge_labels.T)

  ef_p = _seg_sum_ef(dst2, ef, zeros16, n, e)
  agg0_p = _seg_sum_nodes(src2, dst2, h0, zeros16, n, e)
  h1 = _dense_layer(h0, agg0_p, ef_p, W_self0, W_neigh0, w_ef0, b0r)
  agg1_p = _seg_sum_nodes(src2, dst2, h1, zeros16, n, e)
  pooled8 = _dense_layer_pooled(h1, agg1_p, ef_p, W_self1, W_neigh1,
                                w_ef1, b1r)

  pooled = jnp.sum(pooled8, axis=0, keepdims=True)
  return pooled @ W_fc + b_fc


# speedup vs baseline: 13.7665x; 1.1190x over previous
import functools

import jax
import jax.numpy as jnp
from jax import lax
from jax.experimental import pallas as pl
from jax.experimental.pallas import tpu as pltpu
from jax.experimental.pallas import tpu_sc as plsc

NC = 2
NS = 16
NW = NC * NS
CH = 128
WB = 250


KB = 4


def _seg_sum_nodes(src2, dst2, h, zeros16, n_nodes, n_edges):
  n_blocks = n_edges // (CH * KB)
  iters = (n_blocks + NW - 1) // NW
  zr = n_nodes // NS

  mesh = plsc.VectorSubcoreMesh(core_axis_name="c", subcore_axis_name="s")

  @functools.partial(
      pl.kernel,
      out_type=jax.ShapeDtypeStruct((NC, n_nodes, 16), jnp.float32),
      mesh=mesh,
      compiler_params=pltpu.CompilerParams(use_tc_tiling_on_sc=False),
      scratch_types=[
          pltpu.VMEM((2, KB, CH), jnp.int32),
          pltpu.VMEM((2, KB, CH), jnp.int32),
          pltpu.VMEM((2, KB, CH, 16), jnp.float32),
          pltpu.VMEM((WB, 16), jnp.float32),
          pltpu.VMEM_SHARED((n_nodes, 16), jnp.float32),
          pltpu.SemaphoreType.DMA,
          pltpu.SemaphoreType.DMA,
      ],
  )
  def seg_kernel(src_hbm, dst_hbm, h_hbm, z_hbm, out_hbm,
                 src_v, dst_v, rows_v, wb_v, acc_sh, sem_g, sem_s):
    c = lax.axis_index("c")
    s = lax.axis_index("s")
    w = s * NC + c

    pltpu.sync_copy(z_hbm, wb_v)
    for k in range(zr // WB):
      pltpu.sync_copy(wb_v, acc_sh.at[pl.ds(s * zr + k * WB, WB), :])
    plsc.subcore_barrier()

    def body(i, carry):
      b = i * NW + w
      p = i % 2

      @pl.when(b < n_blocks)
      def _():
        pltpu.sync_copy(src_hbm.at[pl.ds(b * KB, KB), :], src_v.at[p])
        pltpu.sync_copy(dst_hbm.at[pl.ds(b * KB, KB), :], dst_v.at[p])
        for j in range(KB):
          pltpu.async_copy(h_hbm.at[src_v.at[p, j]], rows_v.at[p, j], sem_g)
        @pl.when(i >= 1)
        def _():
          for j in range(KB):
            pltpu.make_async_copy(rows_v.at[1 - p, j],
                                  acc_sh.at[dst_v.at[1 - p, j]], sem_s).wait()
        for j in range(KB):
          pltpu.make_async_copy(h_hbm.at[src_v.at[p, j]], rows_v.at[p, j],
                                sem_g).wait()
        for j in range(KB):
          pltpu.async_copy(rows_v.at[p, j], acc_sh.at[dst_v.at[p, j]],
                           sem_s, add=True)

      return carry

    lax.fori_loop(0, iters, body, 0)
    for j in range(KB):
      pltpu.make_async_copy(rows_v.at[0, j], acc_sh.at[dst_v.at[0, j]],
                            sem_s).wait()
    plsc.subcore_barrier()

    for k in range(zr // WB):
      r0 = s * zr + k * WB
      pltpu.sync_copy(acc_sh.at[pl.ds(r0, WB), :], wb_v)
      pltpu.sync_copy(wb_v, out_hbm.at[c, pl.ds(r0, WB), :])

  return seg_kernel(src2, dst2, h, zeros16)


def _pack_edge_features(etT, elT):
  e = etT.shape[1]
  blke = 12800
  grid = e // blke
  pt = jnp.concatenate([jnp.eye(8, dtype=jnp.float32) * (1.0 / 5.0),
                        jnp.zeros((8, 8), jnp.float32)], axis=1)
  plm = jnp.concatenate([jnp.zeros((8, 8), jnp.float32),
                         jnp.eye(8, dtype=jnp.float32)], axis=1)
  dn = (((0,), (0,)), ((), ()))

  def body(t_ref, l_ref, pt_ref, pl_ref, o_ref):
    o_ref[...] = (
        lax.dot_general(t_ref[...], pt_ref[...], dn,
                        preferred_element_type=jnp.float32)
        + lax.dot_general(l_ref[...], pl_ref[...], dn,
                          preferred_element_type=jnp.float32))

  return pl.pallas_call(
      body,
      grid=(grid,),
      in_specs=[pl.BlockSpec((8, blke), lambda i: (0, i)),
                pl.BlockSpec((8, blke), lambda i: (0, i)),
                pl.BlockSpec((8, 16), lambda i: (0, 0)),
                pl.BlockSpec((8, 16), lambda i: (0, 0))],
      out_specs=pl.BlockSpec((blke, 16), lambda i: (i, 0)),
      out_shape=jax.ShapeDtypeStruct((e, 16), jnp.float32),
  )(etT, elT, pt, plm)


def _seg_sum_ef(dst2, ef, zeros16, n_nodes, n_edges):
  n_blocks = n_edges // (CH * KB)
  iters = (n_blocks + NW - 1) // NW
  zr = n_nodes // NS

  mesh = plsc.VectorSubcoreMesh(core_axis_name="c", subcore_axis_name="s")

  @functools.partial(
      pl.kernel,
      out_type=jax.ShapeDtypeStruct((NC, n_nodes, 16), jnp.float32),
      mesh=mesh,
      compiler_params=pltpu.CompilerParams(use_tc_tiling_on_sc=False),
      scratch_types=[
          pltpu.VMEM((2, KB, CH), jnp.int32),
          pltpu.VMEM((2, KB * CH, 16), jnp.float32),
          pltpu.VMEM((WB, 16), jnp.float32),
          pltpu.VMEM_SHARED((n_nodes, 16), jnp.float32),
          pltpu.SemaphoreType.DMA,
      ],
  )
  def ef_kernel(dst_hbm, ef_hbm, z_hbm, out_hbm,
                dst_v, ef_v, wb_v, acc_sh, sem_s):
    c = lax.axis_index("c")
    s = lax.axis_index("s")
    w = s * NC + c

    pltpu.sync_copy(z_hbm, wb_v)
    for k in range(zr // WB):
      pltpu.sync_copy(wb_v, acc_sh.at[pl.ds(s * zr + k * WB, WB), :])
    plsc.subcore_barrier()

    def body(i, carry):
      b = i * NW + w
      p = i % 2

      @pl.when(b < n_blocks)
      def _():
        e0 = b * KB * CH
        pltpu.sync_copy(dst_hbm.at[pl.ds(b * KB, KB), :], dst_v.at[p])
        pltpu.sync_copy(ef_hbm.at[pl.ds(e0, KB * CH), :], ef_v.at[p])

        @pl.when(i >= 1)
        def _():
          for j in range(KB):
            pltpu.make_async_copy(ef_v.at[1 - p, pl.ds(j * CH, CH), :],
                                  acc_sh.at[dst_v.at[1 - p, j]],
                                  sem_s).wait()
        for j in range(KB):
          pltpu.async_copy(ef_v.at[p, pl.ds(j * CH, CH), :],
                           acc_sh.at[dst_v.at[p, j]], sem_s, add=True)

      return carry

    lax.fori_loop(0, iters, body, 0)
    for j in range(KB):
      pltpu.make_async_copy(ef_v.at[0, pl.ds(j * CH, CH), :],
                            acc_sh.at[dst_v.at[0, j]], sem_s).wait()
    plsc.subcore_barrier()

    for k in range(zr // WB):
      r0 = s * zr + k * WB
      pltpu.sync_copy(acc_sh.at[pl.ds(r0, WB), :], wb_v)
      pltpu.sync_copy(wb_v, out_hbm.at[c, pl.ds(r0, WB), :])

  return ef_kernel(dst2, ef, zeros16)


def _split_edge_index(ei):
  e = ei.shape[1]
  blke = 128000
  grid = e // blke
  rb = blke // 128

  def body(ei_ref, s_ref, d_ref):
    x = ei_ref[...]
    s_ref[...] = x[0].reshape(rb, 128)
    d_ref[...] = x[1].reshape(rb, 128)

  return pl.pallas_call(
      body,
      grid=(grid,),
      in_specs=[pl.BlockSpec((2, blke), lambda i: (0, i))],
      out_specs=(pl.BlockSpec((rb, 128), lambda i: (i, 0)),
                 pl.BlockSpec((rb, 128), lambda i: (i, 0))),
      out_shape=(jax.ShapeDtypeStruct((e // 128, 128), jnp.int32),
                 jax.ShapeDtypeStruct((e // 128, 128), jnp.int32)),
  )(ei)


BLK = 5000


def _dense_layer(h, agg_p, ef_p, w_self, w_neigh, w_ef, b):
  n = h.shape[0]
  grid = n // BLK

  def body(h_ref, agg_ref, ef_ref, ws_ref, wn_ref, we_ref, b_ref, out_ref):
    agg = agg_ref[0] + agg_ref[1]
    ef = ef_ref[0] + ef_ref[1]
    acc = jnp.dot(h_ref[...], ws_ref[...], preferred_element_type=jnp.float32)
    acc = acc + jnp.dot(agg, wn_ref[...], preferred_element_type=jnp.float32)
    acc = acc + jnp.dot(ef, we_ref[...], preferred_element_type=jnp.float32)
    out_ref[...] = jnp.maximum(acc + b_ref[...], 0.0)

  return pl.pallas_call(
      body,
      grid=(grid,),
      in_specs=[
          pl.BlockSpec((BLK, 16), lambda i: (i, 0)),
          pl.BlockSpec((NC, BLK, 16), lambda i: (0, i, 0)),
          pl.BlockSpec((NC, BLK, 16), lambda i: (0, i, 0)),
          pl.BlockSpec((16, 16), lambda i: (0, 0)),
          pl.BlockSpec((16, 16), lambda i: (0, 0)),
          pl.BlockSpec((16, 16), lambda i: (0, 0)),
          pl.BlockSpec((1, 16), lambda i: (0, 0)),
      ],
      out_specs=pl.BlockSpec((BLK, 16), lambda i: (i, 0)),
      out_shape=jax.ShapeDtypeStruct((n, 16), jnp.float32),
  )(h, agg_p, ef_p, w_self, w_neigh, w_ef, b)


def _dense_layer_pooled(h, agg_p, ef_p, w_self, w_neigh, w_ef, b):
  n = h.shape[0]
  grid = n // BLK

  def body(h_ref, agg_ref, ef_ref, ws_ref, wn_ref, we_ref, b_ref, out_ref):
    i = pl.program_id(0)
    agg = agg_ref[0] + agg_ref[1]
    ef = ef_ref[0] + ef_ref[1]
    acc = jnp.dot(h_ref[...], ws_ref[...], preferred_element_type=jnp.float32)
    acc = acc + jnp.dot(agg, wn_ref[...], preferred_element_type=jnp.float32)
    acc = acc + jnp.dot(ef, we_ref[...], preferred_element_type=jnp.float32)
    h2 = jnp.maximum(acc + b_ref[...], 0.0)
    part = jnp.sum(h2.reshape(BLK // 8, 8, 16), axis=0)

    @pl.when(i == 0)
    def _():
      out_ref[...] = jnp.zeros_like(out_ref)

    out_ref[...] += part

  return pl.pallas_call(
      body,
      grid=(grid,),
      in_specs=[
          pl.BlockSpec((BLK, 16), lambda i: (i, 0)),
          pl.BlockSpec((NC, BLK, 16), lambda i: (0, i, 0)),
          pl.BlockSpec((NC, BLK, 16), lambda i: (0, i, 0)),
          pl.BlockSpec((16, 16), lambda i: (0, 0)),
          pl.BlockSpec((16, 16), lambda i: (0, 0)),
          pl.BlockSpec((16, 16), lambda i: (0, 0)),
          pl.BlockSpec((1, 16), lambda i: (0, 0)),
      ],
      out_specs=pl.BlockSpec((8, 16), lambda i: (0, 0)),
      out_shape=jax.ShapeDtypeStruct((8, 16), jnp.float32),
  )(h, agg_p, ef_p, w_self, w_neigh, w_ef, b)


def kernel(node_types, node_labels, edge_types, edge_labels, edge_index,
           W_self0, W_neigh0, W_edge0, b0,
           W_self1, W_neigh1, W_edge1, b1,
           W_fc, b_fc):
  n = node_types.shape[0]
  e = edge_types.shape[0]
  src2, dst2 = _split_edge_index(edge_index)

  h0 = jnp.concatenate([node_types * (1.0 / 5.0), node_labels], axis=1)
  w_ef0 = W_edge0
  w_ef1 = W_edge1
  b0r = b0.reshape(1, 16)
  b1r = b1.reshape(1, 16)

  zeros16 = jnp.zeros((WB, 16), jnp.float32)

  ef = _pack_edge_features(edge_types.T, edge_labels.T)

  ef_p = _seg_sum_ef(dst2, ef, zeros16, n, e)
  agg0_p = _seg_sum_nodes(src2, dst2, h0, zeros16, n, e)
  h1 = _dense_layer(h0, agg0_p, ef_p, W_self0, W_neigh0, w_ef0, b0r)
  agg1_p = _seg_sum_nodes(src2, dst2, h1, zeros16, n, e)
  pooled8 = _dense_layer_pooled(h1, agg1_p, ef_p, W_self1, W_neigh1,
                                w_ef1, b1r)

  pooled = jnp.sum(pooled8, axis=0, keepdims=True)
  return pooled @ W_fc + b_fc


# speedup vs baseline: 14.7197x; 1.0692x over previous
import functools

import jax
import jax.numpy as jnp
from jax import lax
from jax.experimental import pallas as pl
from jax.experimental.pallas import tpu as pltpu
from jax.experimental.pallas import tpu_sc as plsc

NC = 2
NS = 16
NW = NC * NS
CH = 128
WB = 250


KB = 4


def _seg_sum_nodes(src2, dst2, h, zeros16, n_nodes, n_edges):
  n_blocks = n_edges // (CH * KB)
  iters = (n_blocks + NW - 1) // NW
  zr = n_nodes // NS

  mesh = plsc.VectorSubcoreMesh(core_axis_name="c", subcore_axis_name="s")

  @functools.partial(
      pl.kernel,
      out_type=jax.ShapeDtypeStruct((NC, n_nodes, 16), jnp.float32),
      mesh=mesh,
      compiler_params=pltpu.CompilerParams(use_tc_tiling_on_sc=False),
      scratch_types=[
          pltpu.VMEM((2, KB, CH), jnp.int32),
          pltpu.VMEM((2, KB, CH), jnp.int32),
          pltpu.VMEM((2, KB, CH, 16), jnp.float32),
          pltpu.VMEM((WB, 16), jnp.float32),
          pltpu.VMEM_SHARED((n_nodes, 16), jnp.float32),
          pltpu.SemaphoreType.DMA,
          pltpu.SemaphoreType.DMA,
      ],
  )
  def seg_kernel(src_hbm, dst_hbm, h_hbm, z_hbm, out_hbm,
                 src_v, dst_v, rows_v, wb_v, acc_sh, sem_g, sem_s):
    c = lax.axis_index("c")
    s = lax.axis_index("s")
    w = s * NC + c

    pltpu.sync_copy(z_hbm, wb_v)
    for k in range(zr // WB):
      pltpu.sync_copy(wb_v, acc_sh.at[pl.ds(s * zr + k * WB, WB), :])
    plsc.subcore_barrier()

    def body(i, carry):
      b = i * NW + w
      p = i % 2

      @pl.when(b < n_blocks)
      def _():
        pltpu.sync_copy(src_hbm.at[pl.ds(b * KB, KB), :], src_v.at[p])
        pltpu.sync_copy(dst_hbm.at[pl.ds(b * KB, KB), :], dst_v.at[p])
        for j in range(KB):
          pltpu.async_copy(h_hbm.at[src_v.at[p, j]], rows_v.at[p, j], sem_g)
        @pl.when(i >= 1)
        def _():
          for j in range(KB):
            pltpu.make_async_copy(rows_v.at[1 - p, j],
                                  acc_sh.at[dst_v.at[1 - p, j]], sem_s).wait()
        for j in range(KB):
          pltpu.make_async_copy(h_hbm.at[src_v.at[p, j]], rows_v.at[p, j],
                                sem_g).wait()
        for j in range(KB):
          pltpu.async_copy(rows_v.at[p, j], acc_sh.at[dst_v.at[p, j]],
                           sem_s, add=True)

      return carry

    lax.fori_loop(0, iters, body, 0)
    for j in range(KB):
      pltpu.make_async_copy(rows_v.at[0, j], acc_sh.at[dst_v.at[0, j]],
                            sem_s).wait()
    plsc.subcore_barrier()

    for k in range(zr // WB):
      r0 = s * zr + k * WB
      pltpu.sync_copy(acc_sh.at[pl.ds(r0, WB), :], wb_v)
      pltpu.sync_copy(wb_v, out_hbm.at[c, pl.ds(r0, WB), :])

  return seg_kernel(src2, dst2, h, zeros16)


def _pack_edge_features(etT, elT):
  e = etT.shape[1]
  blke = 12800
  grid = e // blke
  eye8 = jnp.eye(8, dtype=jnp.float32)
  z8 = jnp.zeros((8, 8), jnp.float32)
  proj = jnp.block([[eye8 * (1.0 / 5.0), z8], [z8, eye8]])
  dn = (((0,), (0,)), ((), ()))

  def body(t_ref, l_ref, p_ref, o_ref):
    x = jnp.concatenate([t_ref[...], l_ref[...]], axis=0)
    o_ref[...] = lax.dot_general(x, p_ref[...], dn,
                                 preferred_element_type=jnp.float32)

  return pl.pallas_call(
      body,
      grid=(grid,),
      in_specs=[pl.BlockSpec((8, blke), lambda i: (0, i)),
                pl.BlockSpec((8, blke), lambda i: (0, i)),
                pl.BlockSpec((16, 16), lambda i: (0, 0))],
      out_specs=pl.BlockSpec((blke, 16), lambda i: (i, 0)),
      out_shape=jax.ShapeDtypeStruct((e, 16), jnp.float32),
  )(etT, elT, proj)


def _seg_sum_ef(dst2, ef, zeros16, n_nodes, n_edges):
  n_blocks = n_edges // (CH * KB)
  iters = (n_blocks + NW - 1) // NW
  zr = n_nodes // NS

  mesh = plsc.VectorSubcoreMesh(core_axis_name="c", subcore_axis_name="s")

  @functools.partial(
      pl.kernel,
      out_type=jax.ShapeDtypeStruct((NC, n_nodes, 16), jnp.float32),
      mesh=mesh,
      compiler_params=pltpu.CompilerParams(use_tc_tiling_on_sc=False),
      scratch_types=[
          pltpu.VMEM((2, KB, CH), jnp.int32),
          pltpu.VMEM((2, KB * CH, 16), jnp.float32),
          pltpu.VMEM((WB, 16), jnp.float32),
          pltpu.VMEM_SHARED((n_nodes, 16), jnp.float32),
          pltpu.SemaphoreType.DMA,
      ],
  )
  def ef_kernel(dst_hbm, ef_hbm, z_hbm, out_hbm,
                dst_v, ef_v, wb_v, acc_sh, sem_s):
    c = lax.axis_index("c")
    s = lax.axis_index("s")
    w = s * NC + c

    pltpu.sync_copy(z_hbm, wb_v)
    for k in range(zr // WB):
      pltpu.sync_copy(wb_v, acc_sh.at[pl.ds(s * zr + k * WB, WB), :])
    plsc.subcore_barrier()

    def body(i, carry):
      b = i * NW + w
      p = i % 2

      @pl.when(b < n_blocks)
      def _():
        e0 = b * KB * CH
        pltpu.sync_copy(dst_hbm.at[pl.ds(b * KB, KB), :], dst_v.at[p])
        pltpu.sync_copy(ef_hbm.at[pl.ds(e0, KB * CH), :], ef_v.at[p])

        @pl.when(i >= 1)
        def _():
          for j in range(KB):
            pltpu.make_async_copy(ef_v.at[1 - p, pl.ds(j * CH, CH), :],
                                  acc_sh.at[dst_v.at[1 - p, j]],
                                  sem_s).wait()
        for j in range(KB):
          pltpu.async_copy(ef_v.at[p, pl.ds(j * CH, CH), :],
                           acc_sh.at[dst_v.at[p, j]], sem_s, add=True)

      return carry

    lax.fori_loop(0, iters, body, 0)
    for j in range(KB):
      pltpu.make_async_copy(ef_v.at[0, pl.ds(j * CH, CH), :],
                            acc_sh.at[dst_v.at[0, j]], sem_s).wait()
    plsc.subcore_barrier()

    for k in range(zr // WB):
      r0 = s * zr + k * WB
      pltpu.sync_copy(acc_sh.at[pl.ds(r0, WB), :], wb_v)
      pltpu.sync_copy(wb_v, out_hbm.at[c, pl.ds(r0, WB), :])

  return ef_kernel(dst2, ef, zeros16)


def _split_edge_index(ei):
  e = ei.shape[1]
  blke = 128000
  grid = e // blke
  rb = blke // 128

  def body(ei_ref, s_ref, d_ref):
    x = ei_ref[...]
    s_ref[...] = x[0].reshape(rb, 128)
    d_ref[...] = x[1].reshape(rb, 128)

  return pl.pallas_call(
      body,
      grid=(grid,),
      in_specs=[pl.BlockSpec((2, blke), lambda i: (0, i))],
      out_specs=(pl.BlockSpec((rb, 128), lambda i: (i, 0)),
                 pl.BlockSpec((rb, 128), lambda i: (i, 0))),
      out_shape=(jax.ShapeDtypeStruct((e // 128, 128), jnp.int32),
                 jax.ShapeDtypeStruct((e // 128, 128), jnp.int32)),
  )(ei)


BLK = 5000


def _dense_layer(h, agg_p, ef_p, w_self, w_neigh, w_ef, b):
  n = h.shape[0]
  grid = n // BLK

  def body(h_ref, agg_ref, ef_ref, ws_ref, wn_ref, we_ref, b_ref, out_ref):
    agg = agg_ref[0] + agg_ref[1]
    ef = ef_ref[0] + ef_ref[1]
    acc = jnp.dot(h_ref[...], ws_ref[...], preferred_element_type=jnp.float32)
    acc = acc + jnp.dot(agg, wn_ref[...], preferred_element_type=jnp.float32)
    acc = acc + jnp.dot(ef, we_ref[...], preferred_element_type=jnp.float32)
    out_ref[...] = jnp.maximum(acc + b_ref[...], 0.0)

  return pl.pallas_call(
      body,
      grid=(grid,),
      in_specs=[
          pl.BlockSpec((BLK, 16), lambda i: (i, 0)),
          pl.BlockSpec((NC, BLK, 16), lambda i: (0, i, 0)),
          pl.BlockSpec((NC, BLK, 16), lambda i: (0, i, 0)),
          pl.BlockSpec((16, 16), lambda i: (0, 0)),
          pl.BlockSpec((16, 16), lambda i: (0, 0)),
          pl.BlockSpec((16, 16), lambda i: (0, 0)),
          pl.BlockSpec((1, 16), lambda i: (0, 0)),
      ],
      out_specs=pl.BlockSpec((BLK, 16), lambda i: (i, 0)),
      out_shape=jax.ShapeDtypeStruct((n, 16), jnp.float32),
  )(h, agg_p, ef_p, w_self, w_neigh, w_ef, b)


def _dense_layer_pooled(h, agg_p, ef_p, w_self, w_neigh, w_ef, b):
  n = h.shape[0]
  grid = n // BLK

  def body(h_ref, agg_ref, ef_ref, ws_ref, wn_ref, we_ref, b_ref, out_ref):
    i = pl.program_id(0)
    agg = agg_ref[0] + agg_ref[1]
    ef = ef_ref[0] + ef_ref[1]
    acc = jnp.dot(h_ref[...], ws_ref[...], preferred_element_type=jnp.float32)
    acc = acc + jnp.dot(agg, wn_ref[...], preferred_element_type=jnp.float32)
    acc = acc + jnp.dot(ef, we_ref[...], preferred_element_type=jnp.float32)
    h2 = jnp.maximum(acc + b_ref[...], 0.0)
    part = jnp.sum(h2.reshape(BLK // 8, 8, 16), axis=0)

    @pl.when(i == 0)
    def _():
      out_ref[...] = jnp.zeros_like(out_ref)

    out_ref[...] += part

  return pl.pallas_call(
      body,
      grid=(grid,),
      in_specs=[
          pl.BlockSpec((BLK, 16), lambda i: (i, 0)),
          pl.BlockSpec((NC, BLK, 16), lambda i: (0, i, 0)),
          pl.BlockSpec((NC, BLK, 16), lambda i: (0, i, 0)),
          pl.BlockSpec((16, 16), lambda i: (0, 0)),
          pl.BlockSpec((16, 16), lambda i: (0, 0)),
          pl.BlockSpec((16, 16), lambda i: (0, 0)),
          pl.BlockSpec((1, 16), lambda i: (0, 0)),
      ],
      out_specs=pl.BlockSpec((8, 16), lambda i: (0, 0)),
      out_shape=jax.ShapeDtypeStruct((8, 16), jnp.float32),
  )(h, agg_p, ef_p, w_self, w_neigh, w_ef, b)


def kernel(node_types, node_labels, edge_types, edge_labels, edge_index,
           W_self0, W_neigh0, W_edge0, b0,
           W_self1, W_neigh1, W_edge1, b1,
           W_fc, b_fc):
  n = node_types.shape[0]
  e = edge_types.shape[0]
  src2, dst2 = _split_edge_index(edge_index)

  h0 = jnp.concatenate([node_types * (1.0 / 5.0), node_labels], axis=1)
  w_ef0 = W_edge0
  w_ef1 = W_edge1
  b0r = b0.reshape(1, 16)
  b1r = b1.reshape(1, 16)

  zeros16 = jnp.zeros((WB, 16), jnp.float32)

  agg0_p = _seg_sum_nodes(src2, dst2, h0, zeros16, n, e)

  ef = _pack_edge_features(edge_types.T, edge_labels.T)
  ef_p = _seg_sum_ef(dst2, ef, zeros16, n, e)
  h1 = _dense_layer(h0, agg0_p, ef_p, W_self0, W_neigh0, w_ef0, b0r)
  agg1_p = _seg_sum_nodes(src2, dst2, h1, zeros16, n, e)
  pooled8 = _dense_layer_pooled(h1, agg1_p, ef_p, W_self1, W_neigh1,
                                w_ef1, b1r)

  pooled = jnp.sum(pooled8, axis=0, keepdims=True)
  return pooled @ W_fc + b_fc
